# baseline (speedup 1.0000x reference)
import functools

import jax
import jax.numpy as jnp
from jax import lax
from jax.experimental import pallas as pl
from jax.experimental.pallas import tpu as pltpu
from jax.experimental.pallas import tpu_sc as plsc

_N1 = 25000
_N2 = 5000
_E0 = 600000
_E1 = 125000
_D_IN = 128
_D_H = 128
_D_OUT = 64

_NC = 2
_NS = 16
_L = 16
_NW = _NC * _NS

_T_HALF = 2500
_T_LOC = 2560
_ROWS_PER_TILE = _T_LOC // _NS

_K = 1024
_NBK = 8
_CAP = 1184

_T_ROWS = 5120
_RPT = _T_ROWS // _NS

_NBLK1 = 32
_EPT1 = _NBLK1 * 128

_PAD_TGT = 1 << 30


def _sc_agg0_body(n_chunks, ept,
                  table_hbm, src_hbm, tgt_hbm, zrow_hbm, zflat_hbm,
                  sums_hbm, cnt_hbm,
                  src_v, tgt_v, fsrc, fltg, bsrc, bidx, grow, cntw,
                  acc_sh, sem):
    cid = lax.axis_index("c")
    sid = lax.axis_index("s")
    lo = cid * _T_HALF
    hi = lo + _T_HALF
    trash = _T_HALF + sid
    lane = lax.iota(jnp.int32, _L)
    onesv = jnp.ones((_L,), jnp.float32)

    r0 = sid * _ROWS_PER_TILE
    pltpu.sync_copy(zrow_hbm, acc_sh.at[pl.ds(r0, 128)])
    pltpu.sync_copy(zrow_hbm.at[pl.ds(0, _ROWS_PER_TILE - 128)],
                    acc_sh.at[pl.ds(r0 + 128, _ROWS_PER_TILE - 128)])
    pltpu.sync_copy(zflat_hbm, cntw)
    plsc.subcore_barrier()

    def drain_block(j):
        for i in range(128 // _L):
            bsrc[0, pl.ds(i * _L, _L)] = fsrc[pl.ds(j * 128 + i * _L, _L)]
            bidx[0, pl.ds(i * _L, _L)] = fltg[pl.ds(j * 128 + i * _L, _L)]
        pltpu.async_copy(table_hbm.at[bsrc.at[0]], grow, sem).wait()
        pltpu.sync_copy(grow, acc_sh.at[bidx.at[0]], add=True)

    def chunk_body(ci, off):
        base = sid * ept + ci * _K
        pltpu.sync_copy(src_hbm.at[pl.ds(base, _K)], src_v)
        pltpu.sync_copy(tgt_hbm.at[pl.ds(base, _K)], tgt_v)

        def scan_body(i, off):
            t = tgt_v[pl.ds(i * _L, _L)]
            s = src_v[pl.ds(i * _L, _L)]
            m = (t >= lo) & (t < hi)
            mi = m.astype(jnp.int32)
            pos = off + plsc.cumsum(mi) - 1
            plsc.store_scatter(fltg, [pos], t - lo, mask=m)
            plsc.store_scatter(fsrc, [pos], s, mask=m)
            cpos = (t - lo) * _L + lane
            plsc.addupdate_scatter(cntw, [cpos], onesv, mask=m)
            return off + jnp.sum(mi)

        off = lax.fori_loop(0, _K // _L, scan_body, off)
        nb = off // 128
        for j in range(_NBK):
            @pl.when(j < nb)
            def _():
                drain_block(j)
        @pl.when(nb > 0)
        def _():
            for i in range(128 // _L):
                vt = fltg[pl.ds(nb * 128 + i * _L, _L)]
                vs = fsrc[pl.ds(nb * 128 + i * _L, _L)]
                fltg[pl.ds(i * _L, _L)] = vt
                fsrc[pl.ds(i * _L, _L)] = vs
        return off - nb * 128

    off = lax.fori_loop(0, n_chunks, chunk_body, 0)

    for i in range(128 // _L):
        fltg[pl.ds(off + i * _L, _L)] = jnp.full((_L,), trash, jnp.int32)
        fsrc[pl.ds(off + i * _L, _L)] = jnp.zeros((_L,), jnp.int32)
    drain_block(0)

    plsc.subcore_barrier()
    out0 = cid * _T_LOC + r0
    pltpu.sync_copy(acc_sh.at[pl.ds(r0, _ROWS_PER_TILE)],
                    sums_hbm.at[pl.ds(out0, _ROWS_PER_TILE)])
    pltpu.sync_copy(cntw, cnt_hbm.at[cid, sid])


def _make_sc_agg0(n_chunks, ept):
    mesh = plsc.VectorSubcoreMesh(core_axis_name="c", subcore_axis_name="s",
                                  num_cores=_NC, num_subcores=_NS)
    return pl.kernel(
        functools.partial(_sc_agg0_body, n_chunks, ept),
        out_type=[
            jax.ShapeDtypeStruct((_NC * _T_LOC, _D_IN), jnp.float32),
            jax.ShapeDtypeStruct((_NC, _NS, _T_LOC * _L), jnp.float32),
        ],
        mesh=mesh,
        scratch_types=[
            pltpu.VMEM((_K,), jnp.int32),
            pltpu.VMEM((_K,), jnp.int32),
            pltpu.VMEM((_CAP,), jnp.int32),
            pltpu.VMEM((_CAP,), jnp.int32),
            pltpu.VMEM((1, 128), jnp.int32),
            pltpu.VMEM((1, 128), jnp.int32),
            pltpu.VMEM((128, _D_IN), jnp.float32),
            pltpu.VMEM((_T_LOC * _L,), jnp.float32),
            pltpu.VMEM_SHARED((_T_LOC, _D_IN), jnp.float32),
            pltpu.SemaphoreType.DMA,
        ],
        compiler_params=pltpu.CompilerParams(needs_layout_passes=False),
    )


def _sc_agg1_body(table_hbm, src_hbm, tgt_hbm, zrow_hbm, sums_hbm,
                  bsrc, bidx, grow0, grow1, acc_sh, gs0, gs1):
    cid = lax.axis_index("c")
    sid = lax.axis_index("s")
    w = cid * _NS + sid
    grows = (grow0, grow1)
    gsems = (gs0, gs1)

    r0 = sid * _RPT
    pltpu.sync_copy(zrow_hbm, acc_sh.at[pl.ds(r0, 128)])
    pltpu.sync_copy(zrow_hbm, acc_sh.at[pl.ds(r0 + 128, 128)])
    pltpu.sync_copy(zrow_hbm.at[pl.ds(0, _RPT - 256)],
                    acc_sh.at[pl.ds(r0 + 256, _RPT - 256)])
    pltpu.sync_copy(src_hbm.at[w], bsrc)
    pltpu.sync_copy(tgt_hbm.at[w], bidx)
    plsc.subcore_barrier()

    def fire(j, buf):
        pltpu.async_copy(table_hbm.at[bsrc.at[j]], grows[buf], gsems[buf])

    def drain(j, buf):
        pltpu.make_async_copy(table_hbm.at[bsrc.at[j]], grows[buf],
                              gsems[buf]).wait()
        pltpu.sync_copy(grows[buf], acc_sh.at[bidx.at[j]], add=True)

    fire(0, 0)

    def pair_body(g, c):
        j0 = 2 * g
        fire(j0 + 1, 1)
        drain(j0, 0)

        @pl.when(j0 + 2 < _NBLK1)
        def _():
            fire(j0 + 2, 0)
        drain(j0 + 1, 1)
        return c

    lax.fori_loop(0, _NBLK1 // 2, pair_body, 0)

    plsc.subcore_barrier()
    pltpu.sync_copy(acc_sh.at[pl.ds(r0, _RPT)],
                    sums_hbm.at[pl.ds(cid * _T_ROWS + r0, _RPT)])


def _make_sc_agg1():
    mesh = plsc.VectorSubcoreMesh(core_axis_name="c", subcore_axis_name="s",
                                  num_cores=_NC, num_subcores=_NS)
    return pl.kernel(
        _sc_agg1_body,
        out_type=jax.ShapeDtypeStruct((_NC * _T_ROWS, _D_H), jnp.float32),
        mesh=mesh,
        scratch_types=[
            pltpu.VMEM((_NBLK1, 128), jnp.int32),
            pltpu.VMEM((_NBLK1, 128), jnp.int32),
            pltpu.VMEM((128, _D_H), jnp.float32),
            pltpu.VMEM((128, _D_H), jnp.float32),
            pltpu.VMEM_SHARED((_T_ROWS, _D_H), jnp.float32),
            pltpu.SemaphoreType.DMA,
            pltpu.SemaphoreType.DMA,
        ],
        compiler_params=pltpu.CompilerParams(needs_layout_passes=False),
    )


_BLK = 256


def _mid_body(sums_ref, cnt_ref, x_ref, wl0_ref, bl0_ref, wr0_ref,
              wl1_ref, cb1_ref, wr1_ref, hw1_ref, hr1_ref):
    cnt = jnp.maximum(cnt_ref[...], 1.0)
    mean = sums_ref[...] / cnt
    h = (jnp.dot(mean, wl0_ref[...], preferred_element_type=jnp.float32)
         + bl0_ref[...]
         + jnp.dot(x_ref[...], wr0_ref[...], preferred_element_type=jnp.float32))
    h = jnp.maximum(h, 0.0)
    hw1_ref[...] = (jnp.dot(h, wl1_ref[...], preferred_element_type=jnp.float32)
                    + cb1_ref[...])
    hr1_ref[...] = jnp.dot(h, wr1_ref[...], preferred_element_type=jnp.float32)


def _final_body(sa_ref, sb_ref, hr1_ref, bl1_ref, out_ref):
    sa = sa_ref[...]
    sb = sb_ref[...]
    cnt = jnp.maximum(sa[:, _D_OUT:_D_OUT + 1] + sb[:, _D_OUT:_D_OUT + 1], 1.0)
    z = ((sa[:, :_D_OUT] + sb[:, :_D_OUT]) / cnt
         + bl1_ref[...] + hr1_ref[...])
    m = jnp.max(z, axis=-1, keepdims=True)
    e = jnp.exp(z - m)
    lse = m + jnp.log(jnp.sum(e, axis=-1, keepdims=True))
    out_ref[...] = z - lse


def _rows_spec(d):
    return pl.BlockSpec((_BLK, d), lambda i: (i, 0))


def _full_spec(shape):
    return pl.BlockSpec(shape, lambda i: (0,) * len(shape))


def _pick_rows(a):
    return jnp.concatenate([a[:_T_HALF], a[_T_LOC:_T_LOC + _T_HALF]], axis=0)


def kernel(x, edge_index_0, edge_index_1, n_id,
           W_l0, b_l0, W_r0, W_l1, b_l1, W_r1):
    del n_id

    f32 = jnp.float32
    nblocks = _T_ROWS // _BLK

    ept0 = _K * ((_E0 // _NS + _K - 1) // _K)
    ep0 = ept0 * _NS
    src0 = jnp.concatenate(
        [edge_index_0[0], jnp.zeros((ep0 - _E0,), jnp.int32)])
    tgt0 = jnp.concatenate(
        [edge_index_0[1], jnp.full((ep0 - _E0,), _PAD_TGT, jnp.int32)])

    ep1 = _EPT1 * _NW
    src1 = jnp.concatenate(
        [edge_index_1[0], jnp.zeros((ep1 - _E1,), jnp.int32)]
    ).reshape(_NW, _NBLK1, 128)
    tgt1 = jnp.concatenate(
        [edge_index_1[1], jnp.full((ep1 - _E1,), _N2, jnp.int32)]
    ).reshape(_NW, _NBLK1, 128)

    zrow128 = jnp.zeros((128, _D_IN), f32)
    zflat = jnp.zeros((_T_LOC * _L,), f32)
    wl1p = jnp.pad(W_l1, ((0, 0), (0, _D_H - _D_OUT)))
    cb1 = jnp.zeros((1, _D_H), f32).at[0, _D_OUT].set(1.0)

    sums0, cnt0 = _make_sc_agg0(ept0 // _K, ept0)(
        x, src0, tgt0, zrow128, zflat)

    sums0p = jnp.pad(_pick_rows(sums0), ((0, _T_ROWS - _N2), (0, 0)))
    c = cnt0.reshape(_NC, _NS, _T_LOC, _L).sum(axis=(1, 3))
    cnt0v = jnp.pad(jnp.concatenate([c[0, :_T_HALF], c[1, :_T_HALF]]),
                    (0, _T_ROWS - _N2), constant_values=1.0)[:, None]
    xp = jnp.pad(x[:_N2], ((0, _T_ROWS - _N2), (0, 0)))

    hw1, hr1 = pl.pallas_call(
        _mid_body,
        grid=(nblocks,),
        in_specs=[
            _rows_spec(_D_IN), _rows_spec(1), _rows_spec(_D_IN),
            _full_spec((_D_IN, _D_H)), _full_spec((1, _D_H)),
            _full_spec((_D_IN, _D_H)),
            _full_spec((_D_H, _D_H)), _full_spec((1, _D_H)),
            _full_spec((_D_H, _D_OUT)),
        ],
        out_specs=[_rows_spec(_D_H), _rows_spec(_D_OUT)],
        out_shape=[
            jax.ShapeDtypeStruct((_T_ROWS, _D_H), f32),
            jax.ShapeDtypeStruct((_T_ROWS, _D_OUT), f32),
        ],
    )(sums0p, cnt0v, xp, W_l0, b_l0.reshape(1, -1), W_r0, wl1p, cb1, W_r1)

    sums1 = _make_sc_agg1()(hw1, src1, tgt1, zrow128)

    outp = pl.pallas_call(
        _final_body,
        grid=(nblocks,),
        in_specs=[
            _rows_spec(_D_H), _rows_spec(_D_H), _rows_spec(_D_OUT),
            _full_spec((1, _D_OUT)),
        ],
        out_specs=_rows_spec(_D_OUT),
        out_shape=jax.ShapeDtypeStruct((_T_ROWS, _D_OUT), f32),
    )(sums1[:_T_ROWS], sums1[_T_ROWS:], hr1, b_l1.reshape(1, -1))

    return outp[:_N2]


# speedup vs baseline: 1.0013x; 1.0013x over previous
import functools

import jax
import jax.numpy as jnp
from jax import lax
from jax.experimental import pallas as pl
from jax.experimental.pallas import tpu as pltpu
from jax.experimental.pallas import tpu_sc as plsc

_N1 = 25000
_N2 = 5000
_E0 = 600000
_E1 = 125000
_D_IN = 128
_D_H = 128
_D_OUT = 64

_NC = 2
_NS = 16
_L = 16
_NW = _NC * _NS

_T_HALF = 2500
_T_LOC = 2560
_ROWS_PER_TILE = _T_LOC // _NS

_K = 1024
_NBK = 8
_CAP = 1184

_T_ROWS = 5120
_RPT = _T_ROWS // _NS

_NBLK1 = 32
_EPT1 = _NBLK1 * 128

_PAD_TGT = 1 << 30


def _sc_agg0_body(n_chunks, ept,
                  table_hbm, src_hbm, tgt_hbm, zrow_hbm, zflat_hbm,
                  sums_hbm, cnt_hbm,
                  src_v, tgt_v, fsrc, fltg, bsrc, bidx, grow, cntw,
                  acc_sh, sem):
    cid = lax.axis_index("c")
    sid = lax.axis_index("s")
    lo = cid * _T_HALF
    hi = lo + _T_HALF
    trash = _T_HALF + sid
    lane = lax.iota(jnp.int32, _L)
    onesv = jnp.ones((_L,), jnp.float32)

    r0 = sid * _ROWS_PER_TILE
    pltpu.sync_copy(zrow_hbm, acc_sh.at[pl.ds(r0, 128)])
    pltpu.sync_copy(zrow_hbm.at[pl.ds(0, _ROWS_PER_TILE - 128)],
                    acc_sh.at[pl.ds(r0 + 128, _ROWS_PER_TILE - 128)])
    pltpu.sync_copy(zflat_hbm, cntw)
    plsc.subcore_barrier()

    def drain_block(j):
        for i in range(128 // _L):
            bsrc[0, pl.ds(i * _L, _L)] = fsrc[pl.ds(j * 128 + i * _L, _L)]
            bidx[0, pl.ds(i * _L, _L)] = fltg[pl.ds(j * 128 + i * _L, _L)]
        pltpu.async_copy(table_hbm.at[bsrc.at[0]], grow, sem).wait()
        pltpu.sync_copy(grow, acc_sh.at[bidx.at[0]], add=True)

    def chunk_body(ci, off):
        base = sid * ept + ci * _K
        pltpu.sync_copy(src_hbm.at[pl.ds(base, _K)], src_v)
        pltpu.sync_copy(tgt_hbm.at[pl.ds(base, _K)], tgt_v)

        def scan_body(i, off):
            t = tgt_v[pl.ds(i * _L, _L)]
            s = src_v[pl.ds(i * _L, _L)]
            m = (t >= lo) & (t < hi)
            mi = m.astype(jnp.int32)
            pos = off + plsc.cumsum(mi) - 1
            plsc.store_scatter(fltg, [pos], t - lo, mask=m)
            plsc.store_scatter(fsrc, [pos], s, mask=m)
            cpos = (t - lo) * _L + lane
            plsc.addupdate_scatter(cntw, [cpos], onesv, mask=m)
            return off + jnp.sum(mi)

        off = lax.fori_loop(0, _K // _L, scan_body, off)
        nb = off // 128
        for j in range(_NBK):
            @pl.when(j < nb)
            def _():
                drain_block(j)
        @pl.when(nb > 0)
        def _():
            for i in range(128 // _L):
                vt = fltg[pl.ds(nb * 128 + i * _L, _L)]
                vs = fsrc[pl.ds(nb * 128 + i * _L, _L)]
                fltg[pl.ds(i * _L, _L)] = vt
                fsrc[pl.ds(i * _L, _L)] = vs
        return off - nb * 128

    off = lax.fori_loop(0, n_chunks, chunk_body, 0)

    for i in range(128 // _L):
        fltg[pl.ds(off + i * _L, _L)] = jnp.full((_L,), trash, jnp.int32)
        fsrc[pl.ds(off + i * _L, _L)] = jnp.zeros((_L,), jnp.int32)
    drain_block(0)

    plsc.subcore_barrier()
    out0 = cid * _T_LOC + r0
    pltpu.sync_copy(acc_sh.at[pl.ds(r0, _ROWS_PER_TILE)],
                    sums_hbm.at[pl.ds(out0, _ROWS_PER_TILE)])
    pltpu.sync_copy(cntw, cnt_hbm.at[cid, sid])


def _make_sc_agg0(n_chunks, ept):
    mesh = plsc.VectorSubcoreMesh(core_axis_name="c", subcore_axis_name="s",
                                  num_cores=_NC, num_subcores=_NS)
    return pl.kernel(
        functools.partial(_sc_agg0_body, n_chunks, ept),
        out_type=[
            jax.ShapeDtypeStruct((_NC * _T_LOC, _D_IN), jnp.float32),
            jax.ShapeDtypeStruct((_NC, _NS, _T_LOC * _L), jnp.float32),
        ],
        mesh=mesh,
        scratch_types=[
            pltpu.VMEM((_K,), jnp.int32),
            pltpu.VMEM((_K,), jnp.int32),
            pltpu.VMEM((_CAP,), jnp.int32),
            pltpu.VMEM((_CAP,), jnp.int32),
            pltpu.VMEM((1, 128), jnp.int32),
            pltpu.VMEM((1, 128), jnp.int32),
            pltpu.VMEM((128, _D_IN), jnp.float32),
            pltpu.VMEM((_T_LOC * _L,), jnp.float32),
            pltpu.VMEM_SHARED((_T_LOC, _D_IN), jnp.float32),
            pltpu.SemaphoreType.DMA,
        ],
        compiler_params=pltpu.CompilerParams(needs_layout_passes=False),
    )


def _sc_agg1_body(table_hbm, src_hbm, tgt_hbm, zrow_hbm, sums_hbm,
                  bsrc, bidx, grow0, grow1, acc_sh, gs0, gs1):
    cid = lax.axis_index("c")
    sid = lax.axis_index("s")
    w = cid * _NS + sid
    grows = (grow0, grow1)
    gsems = (gs0, gs1)

    r0 = sid * _RPT
    pltpu.sync_copy(zrow_hbm, acc_sh.at[pl.ds(r0, 128)])
    pltpu.sync_copy(zrow_hbm, acc_sh.at[pl.ds(r0 + 128, 128)])
    pltpu.sync_copy(zrow_hbm.at[pl.ds(0, _RPT - 256)],
                    acc_sh.at[pl.ds(r0 + 256, _RPT - 256)])
    pltpu.sync_copy(src_hbm.at[w], bsrc)
    pltpu.sync_copy(tgt_hbm.at[w], bidx)
    plsc.subcore_barrier()

    def fire(j, buf):
        pltpu.async_copy(table_hbm.at[bsrc.at[j]], grows[buf], gsems[buf])

    def drain(j, buf):
        pltpu.make_async_copy(table_hbm.at[bsrc.at[j]], grows[buf],
                              gsems[buf]).wait()
        pltpu.sync_copy(grows[buf], acc_sh.at[bidx.at[j]], add=True)

    fire(0, 0)

    def pair_body(g, c):
        j0 = 2 * g
        fire(j0 + 1, 1)
        drain(j0, 0)

        @pl.when(j0 + 2 < _NBLK1)
        def _():
            fire(j0 + 2, 0)
        drain(j0 + 1, 1)
        return c

    lax.fori_loop(0, _NBLK1 // 2, pair_body, 0)

    plsc.subcore_barrier()
    pltpu.sync_copy(acc_sh.at[pl.ds(r0, _RPT)],
                    sums_hbm.at[pl.ds(cid * _T_ROWS + r0, _RPT)])


def _make_sc_agg1():
    mesh = plsc.VectorSubcoreMesh(core_axis_name="c", subcore_axis_name="s",
                                  num_cores=_NC, num_subcores=_NS)
    return pl.kernel(
        _sc_agg1_body,
        out_type=jax.ShapeDtypeStruct((_NC * _T_ROWS, _D_H), jnp.float32),
        mesh=mesh,
        scratch_types=[
            pltpu.VMEM((_NBLK1, 128), jnp.int32),
            pltpu.VMEM((_NBLK1, 128), jnp.int32),
            pltpu.VMEM((128, _D_H), jnp.float32),
            pltpu.VMEM((128, _D_H), jnp.float32),
            pltpu.VMEM_SHARED((_T_ROWS, _D_H), jnp.float32),
            pltpu.SemaphoreType.DMA,
            pltpu.SemaphoreType.DMA,
        ],
        compiler_params=pltpu.CompilerParams(needs_layout_passes=False),
    )


_BLK = 256


def _mid_body(sums_ref, cnt_ref, x_ref, wl0_ref, bl0_ref, wr0_ref,
              wl1_ref, cb1_ref, wr1_ref, hw1_ref, hr1_ref):
    cnt = jnp.maximum(cnt_ref[...], 1.0)
    mean = sums_ref[...] / cnt
    h = (jnp.dot(mean, wl0_ref[...], preferred_element_type=jnp.float32)
         + bl0_ref[...]
         + jnp.dot(x_ref[...], wr0_ref[...], preferred_element_type=jnp.float32))
    h = jnp.maximum(h, 0.0)
    hw1_ref[...] = (jnp.dot(h, wl1_ref[...], preferred_element_type=jnp.float32)
                    + cb1_ref[...])
    hr1_ref[...] = jnp.dot(h, wr1_ref[...], preferred_element_type=jnp.float32)


def _final_body(sa_ref, sb_ref, hr1_ref, bl1_ref, out_ref):
    sa = sa_ref[...]
    sb = sb_ref[...]
    cnt = jnp.maximum(sa[:, _D_OUT:_D_OUT + 1] + sb[:, _D_OUT:_D_OUT + 1], 1.0)
    z = ((sa[:, :_D_OUT] + sb[:, :_D_OUT]) / cnt
         + bl1_ref[...] + hr1_ref[...])
    m = jnp.max(z, axis=-1, keepdims=True)
    e = jnp.exp(z - m)
    lse = m + jnp.log(jnp.sum(e, axis=-1, keepdims=True))
    out_ref[...] = z - lse


def _rows_spec(d):
    return pl.BlockSpec((_BLK, d), lambda i: (i, 0))


def _full_spec(shape):
    return pl.BlockSpec(shape, lambda i: (0,) * len(shape))


def _pick_rows(a):
    return jnp.concatenate([a[:_T_HALF], a[_T_LOC:_T_LOC + _T_HALF]], axis=0)


def kernel(x, edge_index_0, edge_index_1, n_id,
           W_l0, b_l0, W_r0, W_l1, b_l1, W_r1):
    del n_id

    f32 = jnp.float32
    nblocks = _T_ROWS // _BLK

    ept0 = _K * ((_E0 // _NS + _K - 1) // _K)
    ep0 = ept0 * _NS
    src0 = jnp.concatenate(
        [edge_index_0[0], jnp.zeros((ep0 - _E0,), jnp.int32)])
    tgt0 = jnp.concatenate(
        [edge_index_0[1], jnp.full((ep0 - _E0,), _PAD_TGT, jnp.int32)])

    ep1 = _EPT1 * _NW
    src1 = jnp.concatenate(
        [edge_index_1[0], jnp.zeros((ep1 - _E1,), jnp.int32)]
    ).reshape(_NW, _NBLK1, 128)
    pad_tgt1 = _N2 + jnp.arange(ep1 - _E1, dtype=jnp.int32) % (_T_ROWS - _N2)
    tgt1 = jnp.concatenate(
        [edge_index_1[1], pad_tgt1]).reshape(_NW, _NBLK1, 128)

    zrow128 = jnp.zeros((128, _D_IN), f32)
    zflat = jnp.zeros((_T_LOC * _L,), f32)
    wl1p = jnp.pad(W_l1, ((0, 0), (0, _D_H - _D_OUT)))
    cb1 = jnp.zeros((1, _D_H), f32).at[0, _D_OUT].set(1.0)

    sums0, cnt0 = _make_sc_agg0(ept0 // _K, ept0)(
        x, src0, tgt0, zrow128, zflat)

    sums0p = jnp.pad(_pick_rows(sums0), ((0, _T_ROWS - _N2), (0, 0)))
    c = cnt0.reshape(_NC, _NS, _T_LOC, _L).sum(axis=(1, 3))
    cnt0v = jnp.pad(jnp.concatenate([c[0, :_T_HALF], c[1, :_T_HALF]]),
                    (0, _T_ROWS - _N2), constant_values=1.0)[:, None]
    xp = jnp.pad(x[:_N2], ((0, _T_ROWS - _N2), (0, 0)))

    hw1, hr1 = pl.pallas_call(
        _mid_body,
        grid=(nblocks,),
        in_specs=[
            _rows_spec(_D_IN), _rows_spec(1), _rows_spec(_D_IN),
            _full_spec((_D_IN, _D_H)), _full_spec((1, _D_H)),
            _full_spec((_D_IN, _D_H)),
            _full_spec((_D_H, _D_H)), _full_spec((1, _D_H)),
            _full_spec((_D_H, _D_OUT)),
        ],
        out_specs=[_rows_spec(_D_H), _rows_spec(_D_OUT)],
        out_shape=[
            jax.ShapeDtypeStruct((_T_ROWS, _D_H), f32),
            jax.ShapeDtypeStruct((_T_ROWS, _D_OUT), f32),
        ],
    )(sums0p, cnt0v, xp, W_l0, b_l0.reshape(1, -1), W_r0, wl1p, cb1, W_r1)

    sums1 = _make_sc_agg1()(hw1, src1, tgt1, zrow128)

    outp = pl.pallas_call(
        _final_body,
        grid=(nblocks,),
        in_specs=[
            _rows_spec(_D_H), _rows_spec(_D_H), _rows_spec(_D_OUT),
            _full_spec((1, _D_OUT)),
        ],
        out_specs=_rows_spec(_D_OUT),
        out_shape=jax.ShapeDtypeStruct((_T_ROWS, _D_OUT), f32),
    )(sums1[:_T_ROWS], sums1[_T_ROWS:], hr1, b_l1.reshape(1, -1))

    return outp[:_N2]


# speedup vs baseline: 1.9385x; 1.9360x over previous
import functools

import jax
import jax.numpy as jnp
from jax import lax
from jax.experimental import pallas as pl
from jax.experimental.pallas import tpu as pltpu
from jax.experimental.pallas import tpu_sc as plsc

_N1 = 25000
_N2 = 5000
_E0 = 600000
_E1 = 125000
_D_IN = 128
_D_H = 128
_D_OUT = 64

_NC = 2
_NS = 16
_L = 16
_NW = _NC * _NS

_T_HALF = 2500
_T_LOC = 2560
_ROWS_PER_TILE = _T_LOC // _NS

_K = 1024
_NBK = 8
_CAP = 1184

_T_ROWS = 5120
_RPT = _T_ROWS // _NS

_NBLK1 = 32
_EPT1 = _NBLK1 * 128

_PAD_TGT = 1 << 30


def _sc_agg0_body(n_chunks, ept,
                  table_hbm, src_hbm, tgt_hbm, zrow_hbm, zflat_hbm,
                  sums_hbm, cnt_hbm,
                  src_v, tgt_v, fsrc, fltg, bsrc, bidx, grow, cntw,
                  acc_sh, sem):
    cid = lax.axis_index("c")
    sid = lax.axis_index("s")
    lo = cid * _T_HALF
    hi = lo + _T_HALF
    trash = _T_HALF + sid
    lane = lax.iota(jnp.int32, _L)
    onesv = jnp.ones((_L,), jnp.float32)

    r0 = sid * _ROWS_PER_TILE
    pltpu.sync_copy(zrow_hbm, acc_sh.at[pl.ds(r0, 128)])
    pltpu.sync_copy(zrow_hbm.at[pl.ds(0, _ROWS_PER_TILE - 128)],
                    acc_sh.at[pl.ds(r0 + 128, _ROWS_PER_TILE - 128)])
    pltpu.sync_copy(zflat_hbm, cntw)
    plsc.subcore_barrier()

    def drain_block(j):
        for i in range(128 // _L):
            bsrc[0, pl.ds(i * _L, _L)] = fsrc[pl.ds(j * 128 + i * _L, _L)]
            bidx[0, pl.ds(i * _L, _L)] = fltg[pl.ds(j * 128 + i * _L, _L)]
        pltpu.async_copy(table_hbm.at[bsrc.at[0]], grow, sem).wait()
        pltpu.sync_copy(grow, acc_sh.at[bidx.at[0]], add=True)

    def chunk_body(ci, off):
        base = sid * ept + ci * _K
        pltpu.sync_copy(src_hbm.at[pl.ds(base, _K)], src_v)
        pltpu.sync_copy(tgt_hbm.at[pl.ds(base, _K)], tgt_v)

        def scan_body(i, off):
            t = tgt_v[pl.ds(i * _L, _L)]
            s = src_v[pl.ds(i * _L, _L)]
            m = (t >= lo) & (t < hi)
            mi = m.astype(jnp.int32)
            pos = off + plsc.cumsum(mi) - 1
            plsc.store_scatter(fltg, [pos], t - lo, mask=m)
            plsc.store_scatter(fsrc, [pos], s, mask=m)
            cpos = (t - lo) * _L + lane
            plsc.addupdate_scatter(cntw, [cpos], onesv, mask=m)
            return off + jnp.sum(mi)

        off = lax.fori_loop(0, _K // _L, scan_body, off)
        nb = off // 128
        for j in range(_NBK):
            @pl.when(j < nb)
            def _():
                drain_block(j)
        @pl.when(nb > 0)
        def _():
            for i in range(128 // _L):
                vt = fltg[pl.ds(nb * 128 + i * _L, _L)]
                vs = fsrc[pl.ds(nb * 128 + i * _L, _L)]
                fltg[pl.ds(i * _L, _L)] = vt
                fsrc[pl.ds(i * _L, _L)] = vs
        return off - nb * 128

    off = lax.fori_loop(0, n_chunks, chunk_body, 0)

    for i in range(128 // _L):
        fltg[pl.ds(off + i * _L, _L)] = jnp.full((_L,), trash, jnp.int32)
        fsrc[pl.ds(off + i * _L, _L)] = jnp.zeros((_L,), jnp.int32)
    drain_block(0)

    plsc.subcore_barrier()
    out0 = cid * _T_LOC + r0
    pltpu.sync_copy(acc_sh.at[pl.ds(r0, _ROWS_PER_TILE)],
                    sums_hbm.at[pl.ds(out0, _ROWS_PER_TILE)])
    pltpu.sync_copy(cntw, cnt_hbm.at[cid, sid])


def _make_sc_agg0(n_chunks, ept):
    mesh = plsc.VectorSubcoreMesh(core_axis_name="c", subcore_axis_name="s",
                                  num_cores=_NC, num_subcores=_NS)
    return pl.kernel(
        functools.partial(_sc_agg0_body, n_chunks, ept),
        out_type=[
            jax.ShapeDtypeStruct((_NC * _T_LOC, _D_IN), jnp.float32),
            jax.ShapeDtypeStruct((_NC, _NS, _T_LOC * _L), jnp.float32),
        ],
        mesh=mesh,
        scratch_types=[
            pltpu.VMEM((_K,), jnp.int32),
            pltpu.VMEM((_K,), jnp.int32),
            pltpu.VMEM((_CAP,), jnp.int32),
            pltpu.VMEM((_CAP,), jnp.int32),
            pltpu.VMEM((1, 128), jnp.int32),
            pltpu.VMEM((1, 128), jnp.int32),
            pltpu.VMEM((128, _D_IN), jnp.float32),
            pltpu.VMEM((_T_LOC * _L,), jnp.float32),
            pltpu.VMEM_SHARED((_T_LOC, _D_IN), jnp.float32),
            pltpu.SemaphoreType.DMA,
        ],
        compiler_params=pltpu.CompilerParams(needs_layout_passes=False),
    )


def _sc_agg1_body(table_hbm, src_hbm, tgt_hbm, zrow_hbm, sums_hbm,
                  bsrc, bidx, grow0, grow1, acc_sh, gs0, gs1):
    cid = lax.axis_index("c")
    sid = lax.axis_index("s")
    w = cid * _NS + sid
    grows = (grow0, grow1)
    gsems = (gs0, gs1)

    r0 = sid * _RPT
    pltpu.sync_copy(zrow_hbm, acc_sh.at[pl.ds(r0, 128)])
    pltpu.sync_copy(zrow_hbm, acc_sh.at[pl.ds(r0 + 128, 128)])
    pltpu.sync_copy(zrow_hbm.at[pl.ds(0, _RPT - 256)],
                    acc_sh.at[pl.ds(r0 + 256, _RPT - 256)])
    pltpu.sync_copy(src_hbm.at[w], bsrc)
    pltpu.sync_copy(tgt_hbm.at[w], bidx)
    plsc.subcore_barrier()

    def fire(j, buf):
        pltpu.async_copy(table_hbm.at[bsrc.at[j]], grows[buf], gsems[buf])

    def drain(j, buf):
        pltpu.make_async_copy(table_hbm.at[bsrc.at[j]], grows[buf],
                              gsems[buf]).wait()
        pltpu.sync_copy(grows[buf], acc_sh.at[bidx.at[j]], add=True)

    fire(0, 0)

    def pair_body(g, c):
        j0 = 2 * g
        fire(j0 + 1, 1)
        drain(j0, 0)

        @pl.when(j0 + 2 < _NBLK1)
        def _():
            fire(j0 + 2, 0)
        drain(j0 + 1, 1)
        return c

    lax.fori_loop(0, _NBLK1 // 2, pair_body, 0)

    plsc.subcore_barrier()
    pltpu.sync_copy(acc_sh.at[pl.ds(r0, _RPT)],
                    sums_hbm.at[pl.ds(cid * _T_ROWS + r0, _RPT)])


def _make_sc_agg1():
    mesh = plsc.VectorSubcoreMesh(core_axis_name="c", subcore_axis_name="s",
                                  num_cores=_NC, num_subcores=_NS)
    return pl.kernel(
        _sc_agg1_body,
        out_type=jax.ShapeDtypeStruct((_NC * _T_ROWS, _D_H), jnp.float32),
        mesh=mesh,
        scratch_types=[
            pltpu.VMEM((_NBLK1, 128), jnp.int32),
            pltpu.VMEM((_NBLK1, 128), jnp.int32),
            pltpu.VMEM((128, _D_H), jnp.float32),
            pltpu.VMEM((128, _D_H), jnp.float32),
            pltpu.VMEM_SHARED((_T_ROWS, _D_H), jnp.float32),
            pltpu.SemaphoreType.DMA,
            pltpu.SemaphoreType.DMA,
        ],
        compiler_params=pltpu.CompilerParams(needs_layout_passes=False),
    )


_BLK = 256


def _mid_body(sums_ref, cnt_ref, x_ref, wl0_ref, bl0_ref, wr0_ref,
              wl1_ref, cb1_ref, wr1_ref, hw1_ref, hr1_ref):
    cnt = jnp.maximum(cnt_ref[...], 1.0)
    mean = sums_ref[...] / cnt
    h = (jnp.dot(mean, wl0_ref[...], preferred_element_type=jnp.float32)
         + bl0_ref[...]
         + jnp.dot(x_ref[...], wr0_ref[...], preferred_element_type=jnp.float32))
    h = jnp.maximum(h, 0.0)
    hw1_ref[...] = (jnp.dot(h, wl1_ref[...], preferred_element_type=jnp.float32)
                    + cb1_ref[...])
    hr1_ref[...] = jnp.dot(h, wr1_ref[...], preferred_element_type=jnp.float32)


def _final_body(sa_ref, sb_ref, hr1_ref, bl1_ref, out_ref):
    sa = sa_ref[...]
    sb = sb_ref[...]
    cnt = jnp.maximum(sa[:, _D_OUT:_D_OUT + 1] + sb[:, _D_OUT:_D_OUT + 1], 1.0)
    z = ((sa[:, :_D_OUT] + sb[:, :_D_OUT]) / cnt
         + bl1_ref[...] + hr1_ref[...])
    m = jnp.max(z, axis=-1, keepdims=True)
    e = jnp.exp(z - m)
    lse = m + jnp.log(jnp.sum(e, axis=-1, keepdims=True))
    out_ref[...] = z - lse


def _rows_spec(d):
    return pl.BlockSpec((_BLK, d), lambda i: (i, 0))


def _full_spec(shape):
    return pl.BlockSpec(shape, lambda i: (0,) * len(shape))


def _pick_rows(a):
    return jnp.concatenate([a[:_T_HALF], a[_T_LOC:_T_LOC + _T_HALF]], axis=0)


def kernel(x, edge_index_0, edge_index_1, n_id,
           W_l0, b_l0, W_r0, W_l1, b_l1, W_r1):
    del n_id

    f32 = jnp.float32
    nblocks = _T_ROWS // _BLK

    ept0 = _K * ((_E0 // _NS + _K - 1) // _K)
    ep0 = ept0 * _NS
    src0 = jnp.concatenate(
        [edge_index_0[0], jnp.zeros((ep0 - _E0,), jnp.int32)])
    tgt0 = jnp.concatenate(
        [edge_index_0[1], jnp.full((ep0 - _E0,), _PAD_TGT, jnp.int32)])

    ep1 = _EPT1 * _NW
    pad_src1 = jnp.arange(ep1 - _E1, dtype=jnp.int32) % _N2
    src1 = jnp.concatenate(
        [edge_index_1[0], pad_src1]).reshape(_NW, _NBLK1, 128)
    pad_tgt1 = _N2 + jnp.arange(ep1 - _E1, dtype=jnp.int32) % (_T_ROWS - _N2)
    tgt1 = jnp.concatenate(
        [edge_index_1[1], pad_tgt1]).reshape(_NW, _NBLK1, 128)

    zrow128 = jnp.zeros((128, _D_IN), f32)
    zflat = jnp.zeros((_T_LOC * _L,), f32)
    wl1p = jnp.pad(W_l1, ((0, 0), (0, _D_H - _D_OUT)))
    cb1 = jnp.zeros((1, _D_H), f32).at[0, _D_OUT].set(1.0)

    sums0, cnt0 = _make_sc_agg0(ept0 // _K, ept0)(
        x, src0, tgt0, zrow128, zflat)

    sums0p = jnp.pad(_pick_rows(sums0), ((0, _T_ROWS - _N2), (0, 0)))
    c = cnt0.reshape(_NC, _NS, _T_LOC, _L).sum(axis=(1, 3))
    cnt0v = jnp.pad(jnp.concatenate([c[0, :_T_HALF], c[1, :_T_HALF]]),
                    (0, _T_ROWS - _N2), constant_values=1.0)[:, None]
    xp = jnp.pad(x[:_N2], ((0, _T_ROWS - _N2), (0, 0)))

    hw1, hr1 = pl.pallas_call(
        _mid_body,
        grid=(nblocks,),
        in_specs=[
            _rows_spec(_D_IN), _rows_spec(1), _rows_spec(_D_IN),
            _full_spec((_D_IN, _D_H)), _full_spec((1, _D_H)),
            _full_spec((_D_IN, _D_H)),
            _full_spec((_D_H, _D_H)), _full_spec((1, _D_H)),
            _full_spec((_D_H, _D_OUT)),
        ],
        out_specs=[_rows_spec(_D_H), _rows_spec(_D_OUT)],
        out_shape=[
            jax.ShapeDtypeStruct((_T_ROWS, _D_H), f32),
            jax.ShapeDtypeStruct((_T_ROWS, _D_OUT), f32),
        ],
    )(sums0p, cnt0v, xp, W_l0, b_l0.reshape(1, -1), W_r0, wl1p, cb1, W_r1)

    sums1 = _make_sc_agg1()(hw1, src1, tgt1, zrow128)

    outp = pl.pallas_call(
        _final_body,
        grid=(nblocks,),
        in_specs=[
            _rows_spec(_D_H), _rows_spec(_D_H), _rows_spec(_D_OUT),
            _full_spec((1, _D_OUT)),
        ],
        out_specs=_rows_spec(_D_OUT),
        out_shape=jax.ShapeDtypeStruct((_T_ROWS, _D_OUT), f32),
    )(sums1[:_T_ROWS], sums1[_T_ROWS:], hr1, b_l1.reshape(1, -1))

    return outp[:_N2]
